# initial kernel scaffold (unmeasured)
import jax
import jax.numpy as jnp
from jax import lax
from jax.experimental import pallas as pl
from jax.experimental.pallas import tpu as pltpu


def kernel(
    x,
):
    def body(*refs):
        pass

    out_shape = jax.ShapeDtypeStruct(..., jnp.float32)
    return pl.pallas_call(body, out_shape=out_shape)(...)


# baseline (device time: 592942 ns/iter reference)
import jax
import jax.numpy as jnp
from jax import lax
from jax.experimental import pallas as pl
from jax.experimental.pallas import tpu as pltpu

N_DEV = 4


def kernel(x):
    m, n = x.shape
    m_ch = m // N_DEV

    def body(x_ref, out_ref, x_chunk, send_sems, recv_sems, copy_sem):
        my = lax.axis_index("i")
        right = jnp.mod(my + 1, N_DEV)

        def rows(c):
            return pl.ds(c * m_ch, m_ch)

        for h in range(N_DEV - 1):
            send_c = jnp.mod(my - h, N_DEV)
            recv_c = jnp.mod(my - h - 1, N_DEV)
            src = x_ref.at[rows(send_c), :] if h == 0 else out_ref.at[rows(send_c), :]
            rdma = pltpu.make_async_remote_copy(
                src_ref=src,
                dst_ref=out_ref.at[rows(send_c), :],
                send_sem=send_sems.at[h],
                recv_sem=recv_sems.at[h],
                device_id=(right,),
                device_id_type=pl.DeviceIdType.MESH,
            )
            rdma.start()
            cp = pltpu.make_async_copy(x_ref.at[rows(recv_c), :], x_chunk, copy_sem)
            cp.start()
            rdma.wait()
            cp.wait()
            out_ref[rows(recv_c), :] = out_ref[rows(recv_c), :] + x_chunk[:, :]

        for h in range(N_DEV - 1):
            g = N_DEV - 1 + h
            send_c = jnp.mod(my - h + 1, N_DEV)
            rdma = pltpu.make_async_remote_copy(
                src_ref=out_ref.at[rows(send_c), :],
                dst_ref=out_ref.at[rows(send_c), :],
                send_sem=send_sems.at[g],
                recv_sem=recv_sems.at[g],
                device_id=(right,),
                device_id_type=pl.DeviceIdType.MESH,
            )
            rdma.start()
            rdma.wait()

    n_hops = 2 * (N_DEV - 1)
    return pl.pallas_call(
        body,
        out_shape=jax.ShapeDtypeStruct((m, n), x.dtype),
        in_specs=[pl.BlockSpec(memory_space=pl.ANY)],
        out_specs=pl.BlockSpec(memory_space=pltpu.VMEM),
        scratch_shapes=[
            pltpu.VMEM((m_ch, n), x.dtype),
            pltpu.SemaphoreType.DMA((n_hops,)),
            pltpu.SemaphoreType.DMA((n_hops,)),
            pltpu.SemaphoreType.DMA,
        ],
        compiler_params=pltpu.CompilerParams(
            vmem_limit_bytes=60 * 1024 * 1024,
        ),
    )(x)


# device time: 323442 ns/iter; 1.8332x vs baseline; 1.8332x over previous
import jax
import jax.numpy as jnp
from jax import lax
from jax.experimental import pallas as pl
from jax.experimental.pallas import tpu as pltpu

N_DEV = 4


def kernel(x):
    m, n = x.shape
    m_ch = m // N_DEV
    n_half = n // 2

    def body(x_ref, out_ref, x_stage, send_sems, recv_sems, copy_sems):
        my = lax.axis_index("i")
        nbr = [jnp.mod(my + 1, N_DEV), jnp.mod(my - 1, N_DEV)]
        sgn = [1, -1]
        cols = [pl.ds(0, n_half), pl.ds(n_half, n_half)]

        def rows(c):
            return pl.ds(c * m_ch, m_ch)

        for h in range(N_DEV - 1):
            started = []
            for d in range(2):
                send_c = jnp.mod(my - sgn[d] * h, N_DEV)
                recv_c = jnp.mod(my - sgn[d] * (h + 1), N_DEV)
                src_ref = x_ref if h == 0 else out_ref
                rdma = pltpu.make_async_remote_copy(
                    src_ref=src_ref.at[rows(send_c), cols[d]],
                    dst_ref=out_ref.at[rows(send_c), cols[d]],
                    send_sem=send_sems.at[d, h],
                    recv_sem=recv_sems.at[d, h],
                    device_id=(nbr[d],),
                    device_id_type=pl.DeviceIdType.MESH,
                )
                rdma.start()
                cp = pltpu.make_async_copy(
                    x_ref.at[rows(recv_c), cols[d]], x_stage.at[d], copy_sems.at[d]
                )
                cp.start()
                started.append((rdma, cp, recv_c, d))
            for rdma, cp, recv_c, d in started:
                rdma.wait()
                cp.wait()
                out_ref[rows(recv_c), cols[d]] = (
                    out_ref[rows(recv_c), cols[d]] + x_stage[d]
                )

        for h in range(N_DEV - 1):
            g = N_DEV - 1 + h
            started = []
            for d in range(2):
                send_c = jnp.mod(my - sgn[d] * (h - 1), N_DEV)
                rdma = pltpu.make_async_remote_copy(
                    src_ref=out_ref.at[rows(send_c), cols[d]],
                    dst_ref=out_ref.at[rows(send_c), cols[d]],
                    send_sem=send_sems.at[d, g],
                    recv_sem=recv_sems.at[d, g],
                    device_id=(nbr[d],),
                    device_id_type=pl.DeviceIdType.MESH,
                )
                rdma.start()
                started.append(rdma)
            for rdma in started:
                rdma.wait()

    n_hops = 2 * (N_DEV - 1)
    return pl.pallas_call(
        body,
        out_shape=jax.ShapeDtypeStruct((m, n), x.dtype),
        in_specs=[pl.BlockSpec(memory_space=pl.ANY)],
        out_specs=pl.BlockSpec(memory_space=pltpu.VMEM),
        scratch_shapes=[
            pltpu.VMEM((2, m_ch, n_half), x.dtype),
            pltpu.SemaphoreType.DMA((2, n_hops)),
            pltpu.SemaphoreType.DMA((2, n_hops)),
            pltpu.SemaphoreType.DMA((2,)),
        ],
        compiler_params=pltpu.CompilerParams(
            vmem_limit_bytes=60 * 1024 * 1024,
        ),
    )(x)


# device time: 312592 ns/iter; 1.8969x vs baseline; 1.0347x over previous
import jax
import jax.numpy as jnp
from jax import lax
from jax.experimental import pallas as pl
from jax.experimental.pallas import tpu as pltpu

N_DEV = 4
S = 4


def kernel(x):
    m, n = x.shape
    m_ch = m // N_DEV
    n_half = n // 2
    m_sub = m_ch // S
    R = N_DEV - 1
    H = 2 * R

    def body(x_ref, out_ref, x_stage, send_sems, recv_sems, copy_sems):
        my = lax.axis_index("i")
        nbr = [jnp.mod(my + 1, N_DEV), jnp.mod(my - 1, N_DEV)]
        sgn = [1, -1]
        cols = [pl.ds(0, n_half), pl.ds(n_half, n_half)]

        def sub_rows(c, b):
            return pl.ds(c * m_ch + b * m_sub, m_sub)

        def send_chunk(h, d):
            if h < R:
                return jnp.mod(my - sgn[d] * h, N_DEV)
            return jnp.mod(my - sgn[d] * (h - R - 1), N_DEV)

        def recv_chunk(h, d):
            if h < R:
                return jnp.mod(my - sgn[d] * (h + 1), N_DEV)
            return jnp.mod(my - sgn[d] * (h - R), N_DEV)

        def make_send(h, d, b):
            c = send_chunk(h, d)
            src_ref = x_ref if h == 0 else out_ref
            return pltpu.make_async_remote_copy(
                src_ref=src_ref.at[sub_rows(c, b), cols[d]],
                dst_ref=out_ref.at[sub_rows(c, b), cols[d]],
                send_sem=send_sems.at[d, h, b],
                recv_sem=recv_sems.at[d, h, b],
                device_id=(nbr[d],),
                device_id_type=pl.DeviceIdType.MESH,
            )

        def make_recv(h, d, b):
            c = recv_chunk(h, d)
            return pltpu.make_async_remote_copy(
                src_ref=out_ref.at[sub_rows(c, b), cols[d]],
                dst_ref=out_ref.at[sub_rows(c, b), cols[d]],
                send_sem=send_sems.at[d, h, b],
                recv_sem=recv_sems.at[d, h, b],
                device_id=(nbr[d],),
                device_id_type=pl.DeviceIdType.MESH,
            )

        copies = []
        for d in range(2):
            per_d = []
            for g in range(R):
                cp = pltpu.make_async_copy(
                    x_ref.at[pl.ds(recv_chunk(g, d) * m_ch, m_ch), cols[d]],
                    x_stage.at[d, g],
                    copy_sems.at[d, g],
                )
                cp.start()
                per_d.append(cp)
            copies.append(per_d)

        started = []
        for b in range(S):
            for d in range(2):
                rdma = make_send(0, d, b)
                rdma.start()
                started.append(rdma)

        for h in range(1, H + 1):
            g = h - 1
            if g < R:
                for d in range(2):
                    copies[d][g].wait()
            for b in range(S):
                for d in range(2):
                    make_recv(g, d, b).wait_recv()
                    if g < R:
                        rc = recv_chunk(g, d)
                        out_ref[sub_rows(rc, b), cols[d]] = (
                            out_ref[sub_rows(rc, b), cols[d]]
                            + x_stage[d, g, pl.ds(b * m_sub, m_sub), :]
                        )
                    if h < H:
                        rdma = make_send(h, d, b)
                        rdma.start()
                        started.append(rdma)

        for rdma in started:
            rdma.wait_send()

    return pl.pallas_call(
        body,
        out_shape=jax.ShapeDtypeStruct((m, n), x.dtype),
        in_specs=[pl.BlockSpec(memory_space=pl.ANY)],
        out_specs=pl.BlockSpec(memory_space=pltpu.VMEM),
        scratch_shapes=[
            pltpu.VMEM((2, R, m_ch, n_half), x.dtype),
            pltpu.SemaphoreType.DMA((2, H, S)),
            pltpu.SemaphoreType.DMA((2, H, S)),
            pltpu.SemaphoreType.DMA((2, R)),
        ],
        compiler_params=pltpu.CompilerParams(
            vmem_limit_bytes=60 * 1024 * 1024,
        ),
    )(x)


# device time: 303477 ns/iter; 1.9538x vs baseline; 1.0300x over previous
import jax
import jax.numpy as jnp
from jax import lax
from jax.experimental import pallas as pl
from jax.experimental.pallas import tpu as pltpu

N_DEV = 4
S = 4


def kernel(x):
    m, n = x.shape
    m_ch = m // N_DEV
    n_half = n // 2
    m_sub = m_ch // S
    R = N_DEV - 1
    H = 2 * R

    def body(x_ref, out_ref, buf, x_stage, send_sems, recv_sems, copy_sems, out_sems):
        my = lax.axis_index("i")
        nbr = [jnp.mod(my + 1, N_DEV), jnp.mod(my - 1, N_DEV)]
        sgn = [1, -1]
        cols = [pl.ds(0, n_half), pl.ds(n_half, n_half)]

        def sub_rows(c, b):
            return pl.ds(c * m_ch + b * m_sub, m_sub)

        def send_chunk(h, d):
            if h < R:
                return jnp.mod(my - sgn[d] * h, N_DEV)
            return jnp.mod(my - sgn[d] * (h - R - 1), N_DEV)

        def recv_chunk(h, d):
            if h < R:
                return jnp.mod(my - sgn[d] * (h + 1), N_DEV)
            return jnp.mod(my - sgn[d] * (h - R), N_DEV)

        def make_send(h, d, b):
            c = send_chunk(h, d)
            src_ref = x_ref if h == 0 else buf
            return pltpu.make_async_remote_copy(
                src_ref=src_ref.at[sub_rows(c, b), cols[d]],
                dst_ref=buf.at[sub_rows(c, b), cols[d]],
                send_sem=send_sems.at[d, h, b],
                recv_sem=recv_sems.at[d, h, b],
                device_id=(nbr[d],),
                device_id_type=pl.DeviceIdType.MESH,
            )

        def make_recv(h, d, b):
            c = recv_chunk(h, d)
            return pltpu.make_async_remote_copy(
                src_ref=buf.at[sub_rows(c, b), cols[d]],
                dst_ref=buf.at[sub_rows(c, b), cols[d]],
                send_sem=send_sems.at[d, h, b],
                recv_sem=recv_sems.at[d, h, b],
                device_id=(nbr[d],),
                device_id_type=pl.DeviceIdType.MESH,
            )

        out_copies = []

        def flush(c, d, b, g):
            cp = pltpu.make_async_copy(
                buf.at[sub_rows(c, b), cols[d]],
                out_ref.at[sub_rows(c, b), cols[d]],
                out_sems.at[d, g - (R - 1), b],
            )
            cp.start()
            out_copies.append(cp)

        copies = []
        for d in range(2):
            per_d = []
            for g in range(R):
                cp = pltpu.make_async_copy(
                    x_ref.at[pl.ds(recv_chunk(g, d) * m_ch, m_ch), cols[d]],
                    x_stage.at[d, g],
                    copy_sems.at[d, g],
                )
                cp.start()
                per_d.append(cp)
            copies.append(per_d)

        started = []
        for b in range(S):
            for d in range(2):
                rdma = make_send(0, d, b)
                rdma.start()
                started.append(rdma)

        for h in range(1, H + 1):
            g = h - 1
            if g < R:
                for d in range(2):
                    copies[d][g].wait()
            for b in range(S):
                for d in range(2):
                    make_recv(g, d, b).wait_recv()
                    if g < R:
                        rc = recv_chunk(g, d)
                        buf[sub_rows(rc, b), cols[d]] = (
                            buf[sub_rows(rc, b), cols[d]]
                            + x_stage[d, g, pl.ds(b * m_sub, m_sub), :]
                        )
                        if g == R - 1:
                            flush(rc, d, b, g)
                    else:
                        flush(recv_chunk(g, d), d, b, g)
                    if h < H:
                        rdma = make_send(h, d, b)
                        rdma.start()
                        started.append(rdma)

        for cp in out_copies:
            cp.wait()
        for rdma in started:
            rdma.wait_send()

    return pl.pallas_call(
        body,
        out_shape=jax.ShapeDtypeStruct((m, n), x.dtype),
        in_specs=[pl.BlockSpec(memory_space=pl.ANY)],
        out_specs=pl.BlockSpec(memory_space=pl.ANY),
        scratch_shapes=[
            pltpu.VMEM((m, n), x.dtype),
            pltpu.VMEM((2, R, m_ch, n_half), x.dtype),
            pltpu.SemaphoreType.DMA((2, H, S)),
            pltpu.SemaphoreType.DMA((2, H, S)),
            pltpu.SemaphoreType.DMA((2, R)),
            pltpu.SemaphoreType.DMA((2, N_DEV, S)),
        ],
        compiler_params=pltpu.CompilerParams(
            vmem_limit_bytes=60 * 1024 * 1024,
        ),
    )(x)
